# Initial kernel scaffold; baseline (speedup 1.0000x reference)
import jax
import jax.numpy as jnp
from jax.experimental import pallas as pl


def kernel(x, edge_index, batch, W_head, b_head, W1, b1, W2, b2, W3, b3, W_fus, b_fus, W_p1, b_p1, W_p2, b_p2, W_p3, b_p3):
    raise NotImplementedError("write your pallas kernel here")


# speedup vs baseline: 1.0209x; 1.0209x over previous
import functools

import jax
import jax.numpy as jnp
from jax.experimental import pallas as pl
from jax.experimental.pallas import tpu as pltpu

N = 10000
C = 128
K = 16
NCLS = 13

_ROW_TILE = 2000


def _linear2_body(h_ref, wd_ref, ws_ref, b_ref, a_ref, bout_ref):
    h = h_ref[...]
    a_ref[...] = (
        jax.lax.dot_general(h, wd_ref[...], (((1,), (0,)), ((), ())),
                            preferred_element_type=jnp.float32)
        + b_ref[...]
    )
    bout_ref[...] = jax.lax.dot_general(
        h, ws_ref[...], (((1,), (0,)), ((), ())),
        preferred_element_type=jnp.float32)


@jax.jit
def _linear2(h, W, b):
    wd = W[:C] - W[C:]
    ws = W[C:]
    grid = (N // _ROW_TILE,)
    a, bb = pl.pallas_call(
        _linear2_body,
        grid=grid,
        in_specs=[
            pl.BlockSpec((_ROW_TILE, C), lambda i: (i, 0)),
            pl.BlockSpec((C, C), lambda i: (0, 0)),
            pl.BlockSpec((C, C), lambda i: (0, 0)),
            pl.BlockSpec((1, C), lambda i: (0, 0)),
        ],
        out_specs=[
            pl.BlockSpec((_ROW_TILE, C), lambda i: (i, 0)),
            pl.BlockSpec((_ROW_TILE, C), lambda i: (i, 0)),
        ],
        out_shape=[
            jax.ShapeDtypeStruct((N, C), jnp.float32),
            jax.ShapeDtypeStruct((N, C), jnp.float32),
        ],
    )(h, wd, ws, b.reshape(1, C))
    return a, bb


def _tail_body(f0_ref, h1_ref, h2_ref, h3_ref,
               wfus_ref, bfus_ref, wp1a_ref, wp1b_ref, bp1_ref,
               wp2_ref, bp2_ref, wp3_ref, bp3_ref, out_ref):
    feats = jnp.concatenate(
        [f0_ref[...], h1_ref[...], h2_ref[...], h3_ref[...]], axis=1)
    z = jax.lax.dot_general(feats, wfus_ref[...], (((1,), (0,)), ((), ())),
                            preferred_element_type=jnp.float32) + bfus_ref[...]
    fus = jnp.max(jax.nn.relu(z), axis=1, keepdims=True)
    hp = jax.lax.dot_general(feats, wp1a_ref[...], (((1,), (0,)), ((), ())),
                             preferred_element_type=jnp.float32)
    hp = jax.nn.relu(hp + fus * wp1b_ref[...] + bp1_ref[...])
    hp = jax.nn.relu(
        jax.lax.dot_general(hp, wp2_ref[...], (((1,), (0,)), ((), ())),
                            preferred_element_type=jnp.float32) + bp2_ref[...])
    out = jax.lax.dot_general(hp, wp3_ref[...], (((1,), (0,)), ((), ())),
                              preferred_element_type=jnp.float32) + bp3_ref[...]
    out_ref[...] = out


@jax.jit
def _tail(f0, h1, h2, h3, W_fus, b_fus, W_p1, b_p1, W_p2, b_p2, W_p3, b_p3):
    wp1a = W_p1[:4 * C]
    wp1b = W_p1[4 * C:]
    wp3 = jnp.zeros((256, 128), jnp.float32).at[:, :NCLS].set(W_p3)
    bp3 = jnp.zeros((1, 128), jnp.float32).at[:, :NCLS].set(b_p3)
    grid = (N // _ROW_TILE,)
    out = pl.pallas_call(
        _tail_body,
        grid=grid,
        in_specs=[
            pl.BlockSpec((_ROW_TILE, C), lambda i: (i, 0)),
            pl.BlockSpec((_ROW_TILE, C), lambda i: (i, 0)),
            pl.BlockSpec((_ROW_TILE, C), lambda i: (i, 0)),
            pl.BlockSpec((_ROW_TILE, C), lambda i: (i, 0)),
            pl.BlockSpec((4 * C, 1024), lambda i: (0, 0)),
            pl.BlockSpec((1, 1024), lambda i: (0, 0)),
            pl.BlockSpec((4 * C, 4 * C), lambda i: (0, 0)),
            pl.BlockSpec((1, 4 * C), lambda i: (0, 0)),
            pl.BlockSpec((1, 4 * C), lambda i: (0, 0)),
            pl.BlockSpec((4 * C, 256), lambda i: (0, 0)),
            pl.BlockSpec((1, 256), lambda i: (0, 0)),
            pl.BlockSpec((256, 128), lambda i: (0, 0)),
            pl.BlockSpec((1, 128), lambda i: (0, 0)),
        ],
        out_specs=pl.BlockSpec((_ROW_TILE, 128), lambda i: (i, 0)),
        out_shape=jax.ShapeDtypeStruct((N, 128), jnp.float32),
    )(f0, h1, h2, h3, W_fus, b_fus.reshape(1, -1), wp1a, wp1b,
      b_p1.reshape(1, -1), W_p2, b_p2.reshape(1, -1), wp3, bp3)
    return out[:, :NCLS]


def _knn_idx(h, batch, k, dilation):
    x2 = jnp.sum(h * h, axis=1)
    d2 = x2[:, None] + x2[None, :] - 2.0 * (h @ h.T)
    big = jnp.float32(1e10)
    d2 = jnp.where(batch[:, None] != batch[None, :], big, d2)
    r = jnp.arange(N)
    d2 = d2.at[r, r].set(big)
    _, idx = jax.lax.top_k(-d2, k * dilation)
    return idx[:, ::dilation]


def kernel(x, edge_index, batch, W_head, b_head, W1, b1, W2, b2, W3, b3,
           W_fus, b_fus, W_p1, b_p1, W_p2, b_p2, W_p3, b_p3):
    a0, b0 = _linear2(x, W_head, b_head)
    src, dst = edge_index[0], edge_index[1]
    m = jax.ops.segment_max(b0[src], dst, num_segments=N)
    f0 = jax.nn.relu(a0 + m)
    f0 = jnp.where(jnp.isneginf(m), 0.0, f0)

    h = f0
    feats = [f0]
    for i, (W, b) in enumerate([(W1, b1), (W2, b2), (W3, b3)]):
        dilation = i % 8 + 1
        idx = _knn_idx(h, batch, K, dilation)
        a, bv = _linear2(h, W, b)
        maxb = jnp.max(bv[idx], axis=1)
        h = jax.nn.relu(a + maxb) + h
        feats.append(h)

    return _tail(feats[0], feats[1], feats[2], feats[3],
                 W_fus, b_fus, W_p1, b_p1, W_p2, b_p2, W_p3, b_p3)


# speedup vs baseline: 9.5185x; 9.3241x over previous
import jax
import jax.numpy as jnp
from jax.experimental import pallas as pl
from jax.experimental.pallas import tpu as pltpu

N = 10000
NPAD = 10240
C = 128
K = 16
NCLS = 13
CHUNK = 16
NCHUNK = NPAD // CHUNK
BIG = 1e10
INF = float('inf')

_CT = 2048
_RT = 2048
_ET = 4000
_NT = 1000
_TT = 2000


def _edge_m_body(xi_ref, xj_ref, w_ref, b_ref, m_ref):
    xi = xi_ref[...]
    xj = xj_ref[...]
    u = jnp.concatenate([xi, xj - xi], axis=1)
    m_ref[...] = jax.nn.relu(
        jax.lax.dot_general(u, w_ref[...], (((1,), (0,)), ((), ())),
                            preferred_element_type=jnp.float32)
        + b_ref[...])


def _edge_m(xi, xj, W, b):
    E = xi.shape[0]
    return pl.pallas_call(
        _edge_m_body,
        grid=(E // _ET,),
        in_specs=[
            pl.BlockSpec((_ET, C), lambda i: (i, 0)),
            pl.BlockSpec((_ET, C), lambda i: (i, 0)),
            pl.BlockSpec((2 * C, C), lambda i: (0, 0)),
            pl.BlockSpec((1, C), lambda i: (0, 0)),
        ],
        out_specs=pl.BlockSpec((_ET, C), lambda i: (i, 0)),
        out_shape=jax.ShapeDtypeStruct((E, C), jnp.float32),
    )(xi, xj, W, b.reshape(1, C))


def _dense_conv_body(h_ref, xj_ref, w_ref, b_ref, o_ref):
    h = h_ref[...]
    xi = jnp.broadcast_to(h[:, None, :], (_NT, K, C)).reshape(_NT * K, C)
    xj = xj_ref[...]
    u = jnp.concatenate([xi, xj - xi], axis=1)
    m = jax.nn.relu(
        jax.lax.dot_general(u, w_ref[...], (((1,), (0,)), ((), ())),
                            preferred_element_type=jnp.float32)
        + b_ref[...])
    o_ref[...] = jnp.max(m.reshape(_NT, K, C), axis=1) + h


def _dense_conv(h, xj_flat, W, b):
    return pl.pallas_call(
        _dense_conv_body,
        grid=(N // _NT,),
        in_specs=[
            pl.BlockSpec((_NT, C), lambda i: (i, 0)),
            pl.BlockSpec((_NT * K, C), lambda i: (i, 0)),
            pl.BlockSpec((2 * C, C), lambda i: (0, 0)),
            pl.BlockSpec((1, C), lambda i: (0, 0)),
        ],
        out_specs=pl.BlockSpec((_NT, C), lambda i: (i, 0)),
        out_shape=jax.ShapeDtypeStruct((N, C), jnp.float32),
    )(h, xj_flat, W, b.reshape(1, C))


def _gram_body(hc_ref, hr_ref, x2c_ref, x2r_ref, bc_ref, br_ref,
               dt_ref, c_ref):
    i = pl.program_id(0)
    j = pl.program_id(1)
    g = jax.lax.dot_general(hc_ref[...], hr_ref[...],
                            (((1,), (1,)), ((), ())),
                            preferred_element_type=jnp.float32)
    d2 = x2c_ref[...] + x2r_ref[...] - 2.0 * g
    colg = jax.lax.broadcasted_iota(jnp.int32, (_CT, _RT), 0) + i * _CT
    rowg = jax.lax.broadcasted_iota(jnp.int32, (_CT, _RT), 1) + j * _RT
    mask = (bc_ref[...] != br_ref[...]) | (colg == rowg) | (colg >= N)
    d2 = jnp.where(mask, BIG, d2)
    d2 = jnp.where(colg >= N, INF, d2)
    dt_ref[...] = d2
    c_ref[...] = jnp.min(d2.reshape(_CT // CHUNK, CHUNK, _RT), axis=1)


def _gram_chunkmin(h_pad, x2_pad, batch_pad):
    grid = (NPAD // _CT, NPAD // _RT)
    x2c = x2_pad.reshape(NPAD, 1)
    x2r = x2_pad.reshape(1, NPAD)
    bc = batch_pad.reshape(NPAD, 1)
    br = batch_pad.reshape(1, NPAD)
    dt, cmin = pl.pallas_call(
        _gram_body,
        grid=grid,
        in_specs=[
            pl.BlockSpec((_CT, C), lambda i, j: (i, 0)),
            pl.BlockSpec((_RT, C), lambda i, j: (j, 0)),
            pl.BlockSpec((_CT, 1), lambda i, j: (i, 0)),
            pl.BlockSpec((1, _RT), lambda i, j: (0, j)),
            pl.BlockSpec((_CT, 1), lambda i, j: (i, 0)),
            pl.BlockSpec((1, _RT), lambda i, j: (0, j)),
        ],
        out_specs=[
            pl.BlockSpec((_CT, _RT), lambda i, j: (i, j)),
            pl.BlockSpec((_CT // CHUNK, _RT), lambda i, j: (i, j)),
        ],
        out_shape=[
            jax.ShapeDtypeStruct((NPAD, NPAD), jnp.float32),
            jax.ShapeDtypeStruct((NCHUNK, NPAD), jnp.float32),
        ],
    )(h_pad, h_pad, x2c, x2r, bc, br)
    return dt, cmin


def _phase_a_body(ka, ka_pad, c_ref, o_ref):
    c = c_ref[...]
    sub = jax.lax.broadcasted_iota(jnp.int32, (NCHUNK, _RT), 0)
    ids = []
    for _ in range(ka):
        m = jnp.min(c, axis=0, keepdims=True)
        eq = c == m
        cid = jnp.min(jnp.where(eq, sub, NCHUNK), axis=0, keepdims=True)
        ids.append(cid)
        c = jnp.where(sub == cid, INF, c)
    for _ in range(ka_pad - ka):
        ids.append(ids[-1])
    o_ref[...] = jnp.concatenate(ids, axis=0)


def _phase_a(cmin, ka):
    ka_pad = (ka + 7) // 8 * 8
    import functools
    body = functools.partial(_phase_a_body, ka, ka_pad)
    return pl.pallas_call(
        body,
        grid=(NPAD // _RT,),
        in_specs=[pl.BlockSpec((NCHUNK, _RT), lambda i: (0, i))],
        out_specs=pl.BlockSpec((ka_pad, _RT), lambda i: (0, i)),
        out_shape=jax.ShapeDtypeStruct((ka_pad, NPAD), jnp.int32),
    )(cmin)


def _phase_c_body(ncand, kd, dil, v_ref, col_ref, o_ref):
    v = v_ref[...]
    cols = col_ref[...]
    out = []
    for r in range(kd):
        m = jnp.min(v, axis=0, keepdims=True)
        eq = v == m
        col = jnp.min(jnp.where(eq, cols, NPAD), axis=0, keepdims=True)
        if r % dil == 0:
            out.append(col)
        v = jnp.where(cols == col, INF, v)
    o_ref[...] = jnp.concatenate(out, axis=0)


def _phase_c(cand, cand_cols, kd, dil):
    ncand = cand.shape[0]
    import functools
    body = functools.partial(_phase_c_body, ncand, kd, dil)
    return pl.pallas_call(
        body,
        grid=(NPAD // _RT,),
        in_specs=[
            pl.BlockSpec((ncand, _RT), lambda i: (0, i)),
            pl.BlockSpec((ncand, _RT), lambda i: (0, i)),
        ],
        out_specs=pl.BlockSpec((K, _RT), lambda i: (0, i)),
        out_shape=jax.ShapeDtypeStruct((K, NPAD), jnp.int32),
    )(cand, cand_cols)


def _knn_idx(h, x2, batch_pad, dilation):
    kd = (K - 1) * dilation + 1
    h_pad = jnp.pad(h, ((0, NPAD - N), (0, 0)))
    x2_pad = jnp.pad(x2, (0, NPAD - N))
    dt, cmin = _gram_chunkmin(h_pad, x2_pad, batch_pad)
    ids = _phase_a(cmin, kd)[:kd]
    cand_cols = (ids[:, None, :] * CHUNK
                 + jnp.arange(CHUNK, dtype=jnp.int32)[None, :, None]
                 ).reshape(kd * CHUNK, NPAD)
    cand = jnp.take_along_axis(dt, cand_cols, axis=0)
    idx_t = _phase_c(cand, cand_cols, kd, dilation)
    return idx_t[:, :N].T


def _tail_body(f0_ref, h1_ref, h2_ref, h3_ref,
               wfus_ref, bfus_ref, wp1a_ref, wp1b_ref, bp1_ref,
               wp2_ref, bp2_ref, wp3_ref, bp3_ref, out_ref):
    feats = jnp.concatenate(
        [f0_ref[...], h1_ref[...], h2_ref[...], h3_ref[...]], axis=1)
    z = jax.lax.dot_general(feats, wfus_ref[...], (((1,), (0,)), ((), ())),
                            preferred_element_type=jnp.float32) + bfus_ref[...]
    fus = jnp.max(jax.nn.relu(z), axis=1, keepdims=True)
    hp = jax.lax.dot_general(feats, wp1a_ref[...], (((1,), (0,)), ((), ())),
                             preferred_element_type=jnp.float32)
    hp = jax.nn.relu(hp + fus * wp1b_ref[...] + bp1_ref[...])
    hp = jax.nn.relu(
        jax.lax.dot_general(hp, wp2_ref[...], (((1,), (0,)), ((), ())),
                            preferred_element_type=jnp.float32) + bp2_ref[...])
    out_ref[...] = jax.lax.dot_general(
        hp, wp3_ref[...], (((1,), (0,)), ((), ())),
        preferred_element_type=jnp.float32) + bp3_ref[...]


def _tail(f0, h1, h2, h3, W_fus, b_fus, W_p1, b_p1, W_p2, b_p2, W_p3, b_p3):
    wp1a = W_p1[:4 * C]
    wp1b = W_p1[4 * C:]
    wp3 = jnp.zeros((256, 128), jnp.float32).at[:, :NCLS].set(W_p3)
    bp3 = jnp.zeros((1, 128), jnp.float32).at[:, :NCLS].set(b_p3)
    out = pl.pallas_call(
        _tail_body,
        grid=(N // _TT,),
        in_specs=[
            pl.BlockSpec((_TT, C), lambda i: (i, 0)),
            pl.BlockSpec((_TT, C), lambda i: (i, 0)),
            pl.BlockSpec((_TT, C), lambda i: (i, 0)),
            pl.BlockSpec((_TT, C), lambda i: (i, 0)),
            pl.BlockSpec((4 * C, 1024), lambda i: (0, 0)),
            pl.BlockSpec((1, 1024), lambda i: (0, 0)),
            pl.BlockSpec((4 * C, 4 * C), lambda i: (0, 0)),
            pl.BlockSpec((1, 4 * C), lambda i: (0, 0)),
            pl.BlockSpec((1, 4 * C), lambda i: (0, 0)),
            pl.BlockSpec((4 * C, 256), lambda i: (0, 0)),
            pl.BlockSpec((1, 256), lambda i: (0, 0)),
            pl.BlockSpec((256, 128), lambda i: (0, 0)),
            pl.BlockSpec((1, 128), lambda i: (0, 0)),
        ],
        out_specs=pl.BlockSpec((_TT, 128), lambda i: (i, 0)),
        out_shape=jax.ShapeDtypeStruct((N, 128), jnp.float32),
    )(f0, h1, h2, h3, W_fus, b_fus.reshape(1, -1), wp1a, wp1b,
      b_p1.reshape(1, -1), W_p2, b_p2.reshape(1, -1), wp3, bp3)
    return out[:, :NCLS]


def kernel(x, edge_index, batch, W_head, b_head, W1, b1, W2, b2, W3, b3,
           W_fus, b_fus, W_p1, b_p1, W_p2, b_p2, W_p3, b_p3):
    src, dst = edge_index[0], edge_index[1]
    m = _edge_m(x[dst], x[src], W_head, b_head)
    agg = jax.ops.segment_max(m, dst, num_segments=N)
    f0 = jnp.where(jnp.isfinite(agg), agg, 0.0)

    batch_pad = jnp.pad(batch, (0, NPAD - N), constant_values=-1)
    h = f0
    feats = [f0]
    for i, (W, b) in enumerate([(W1, b1), (W2, b2), (W3, b3)]):
        dilation = i % 8 + 1
        x2 = jnp.sum(h * h, axis=1)
        idx = _knn_idx(h, x2, batch_pad, dilation)
        xj = h[idx.reshape(-1)]
        h = _dense_conv(h, xj, W, b)
        feats.append(h)

    return _tail(feats[0], feats[1], feats[2], feats[3],
                 W_fus, b_fus, W_p1, b_p1, W_p2, b_p2, W_p3, b_p3)


# speedup vs baseline: 10.3383x; 1.0861x over previous
import functools

import jax
import jax.numpy as jnp
from jax.experimental import pallas as pl
from jax.experimental.pallas import tpu as pltpu
from jax.experimental.pallas import tpu_sc as plsc

N = 10000
NPAD = 10240
C = 128
K = 16
NCLS = 13
CHUNK = 16
NCHUNK = NPAD // CHUNK
BIG = 1e10
INF = float('inf')

_CT = 2048
_RT = 2048
_ET = 4000
_NT = 1000
_TT = 2000


_SC_NW = 32
_SC_CHUNK = 128
_SC_NBUF = 4


def _sc_gather_rows(table, idx):
    B = idx.shape[0]
    D = table.shape[1]
    b_per_w = B // _SC_NW
    nch = b_per_w // _SC_CHUNK
    mesh = plsc.VectorSubcoreMesh(core_axis_name="c", subcore_axis_name="s")

    @functools.partial(
        pl.kernel, mesh=mesh,
        out_type=jax.ShapeDtypeStruct((B, D), table.dtype),
        scratch_types=[
            pltpu.VMEM((b_per_w,), jnp.int32),
            pltpu.VMEM((_SC_NBUF, _SC_CHUNK, D), jnp.float32),
            pltpu.SemaphoreType.DMA,
            pltpu.SemaphoreType.DMA((_SC_NBUF,)),
        ],
    )
    def k(table_hbm, idx_hbm, out_hbm, idx_v, rows_v, sem, gsems):
        wid = jax.lax.axis_index("s") * 2 + jax.lax.axis_index("c")
        base = wid * b_per_w
        pltpu.async_copy(idx_hbm.at[pl.ds(base, b_per_w)], idx_v, sem).wait()

        def issue(ci, b):
            return pltpu.async_copy(
                table_hbm.at[idx_v.at[pl.ds(ci * _SC_CHUNK, _SC_CHUNK)]],
                rows_v.at[b], gsems.at[b])

        for b in range(_SC_NBUF):
            issue(b, b)


        @pl.loop(0, nch, step=_SC_NBUF)
        def _(c0):
            for b in range(_SC_NBUF):
                ci = c0 + b
                pltpu.make_async_copy(
                    table_hbm.at[idx_v.at[pl.ds(0, _SC_CHUNK)]],
                    rows_v.at[b], gsems.at[b]).wait()
                pltpu.sync_copy(
                    rows_v.at[b],
                    out_hbm.at[pl.ds(base + ci * _SC_CHUNK, _SC_CHUNK)])
                nxt = ci + _SC_NBUF

                @pl.when(nxt < nch)
                def _():
                    issue(nxt, b)

    return k(table, idx)


def _edge_m_body(xi_ref, xj_ref, w_ref, b_ref, m_ref):
    xi = xi_ref[...]
    xj = xj_ref[...]
    u = jnp.concatenate([xi, xj - xi], axis=1)
    m_ref[...] = jax.nn.relu(
        jax.lax.dot_general(u, w_ref[...], (((1,), (0,)), ((), ())),
                            preferred_element_type=jnp.float32)
        + b_ref[...])


def _edge_m(xi, xj, W, b):
    E = xi.shape[0]
    return pl.pallas_call(
        _edge_m_body,
        grid=(E // _ET,),
        in_specs=[
            pl.BlockSpec((_ET, C), lambda i: (i, 0)),
            pl.BlockSpec((_ET, C), lambda i: (i, 0)),
            pl.BlockSpec((2 * C, C), lambda i: (0, 0)),
            pl.BlockSpec((1, C), lambda i: (0, 0)),
        ],
        out_specs=pl.BlockSpec((_ET, C), lambda i: (i, 0)),
        out_shape=jax.ShapeDtypeStruct((E, C), jnp.float32),
    )(xi, xj, W, b.reshape(1, C))


def _dense_conv_body(h_ref, xj_ref, w_ref, b_ref, o_ref):
    h = h_ref[...]
    xi = jnp.broadcast_to(h[:, None, :], (_NT, K, C)).reshape(_NT * K, C)
    xj = xj_ref[...]
    u = jnp.concatenate([xi, xj - xi], axis=1)
    m = jax.nn.relu(
        jax.lax.dot_general(u, w_ref[...], (((1,), (0,)), ((), ())),
                            preferred_element_type=jnp.float32)
        + b_ref[...])
    o_ref[...] = jnp.max(m.reshape(_NT, K, C), axis=1) + h


def _dense_conv(h, xj_flat, W, b):
    return pl.pallas_call(
        _dense_conv_body,
        grid=(N // _NT,),
        in_specs=[
            pl.BlockSpec((_NT, C), lambda i: (i, 0)),
            pl.BlockSpec((_NT * K, C), lambda i: (i, 0)),
            pl.BlockSpec((2 * C, C), lambda i: (0, 0)),
            pl.BlockSpec((1, C), lambda i: (0, 0)),
        ],
        out_specs=pl.BlockSpec((_NT, C), lambda i: (i, 0)),
        out_shape=jax.ShapeDtypeStruct((N, C), jnp.float32),
    )(h, xj_flat, W, b.reshape(1, C))


def _gram_body(hc_ref, hr_ref, x2c_ref, x2r_ref, bc_ref, br_ref,
               dt_ref, c_ref):
    i = pl.program_id(0)
    j = pl.program_id(1)
    g = jax.lax.dot_general(hc_ref[...], hr_ref[...],
                            (((1,), (1,)), ((), ())),
                            preferred_element_type=jnp.float32)
    d2 = x2c_ref[...] + x2r_ref[...] - 2.0 * g
    colg = jax.lax.broadcasted_iota(jnp.int32, (_CT, _RT), 0) + i * _CT
    rowg = jax.lax.broadcasted_iota(jnp.int32, (_CT, _RT), 1) + j * _RT
    mask = (bc_ref[...] != br_ref[...]) | (colg == rowg) | (colg >= N)
    d2 = jnp.where(mask, BIG, d2)
    d2 = jnp.where(colg >= N, INF, d2)
    dt_ref[...] = d2
    c_ref[...] = jnp.min(d2.reshape(_CT // CHUNK, CHUNK, _RT), axis=1)


def _gram_chunkmin(h_pad, x2_pad, batch_pad):
    grid = (NPAD // _CT, NPAD // _RT)
    x2c = x2_pad.reshape(NPAD, 1)
    x2r = x2_pad.reshape(1, NPAD)
    bc = batch_pad.reshape(NPAD, 1)
    br = batch_pad.reshape(1, NPAD)
    dt, cmin = pl.pallas_call(
        _gram_body,
        grid=grid,
        in_specs=[
            pl.BlockSpec((_CT, C), lambda i, j: (i, 0)),
            pl.BlockSpec((_RT, C), lambda i, j: (j, 0)),
            pl.BlockSpec((_CT, 1), lambda i, j: (i, 0)),
            pl.BlockSpec((1, _RT), lambda i, j: (0, j)),
            pl.BlockSpec((_CT, 1), lambda i, j: (i, 0)),
            pl.BlockSpec((1, _RT), lambda i, j: (0, j)),
        ],
        out_specs=[
            pl.BlockSpec((_CT, _RT), lambda i, j: (i, j)),
            pl.BlockSpec((_CT // CHUNK, _RT), lambda i, j: (i, j)),
        ],
        out_shape=[
            jax.ShapeDtypeStruct((NPAD, NPAD), jnp.float32),
            jax.ShapeDtypeStruct((NCHUNK, NPAD), jnp.float32),
        ],
    )(h_pad, h_pad, x2c, x2r, bc, br)
    return dt, cmin


def _phase_a_body(ka, ka_pad, c_ref, o_ref):
    c = c_ref[...]
    sub = jax.lax.broadcasted_iota(jnp.int32, (NCHUNK, _RT), 0)
    ids = []
    for _ in range(ka):
        m = jnp.min(c, axis=0, keepdims=True)
        eq = c == m
        cid = jnp.min(jnp.where(eq, sub, NCHUNK), axis=0, keepdims=True)
        ids.append(cid)
        c = jnp.where(sub == cid, INF, c)
    for _ in range(ka_pad - ka):
        ids.append(ids[-1])
    o_ref[...] = jnp.concatenate(ids, axis=0)


def _phase_a(cmin, ka):
    ka_pad = (ka + 7) // 8 * 8
    import functools
    body = functools.partial(_phase_a_body, ka, ka_pad)
    return pl.pallas_call(
        body,
        grid=(NPAD // _RT,),
        in_specs=[pl.BlockSpec((NCHUNK, _RT), lambda i: (0, i))],
        out_specs=pl.BlockSpec((ka_pad, _RT), lambda i: (0, i)),
        out_shape=jax.ShapeDtypeStruct((ka_pad, NPAD), jnp.int32),
    )(cmin)


def _phase_c_body(ncand, kd, dil, v_ref, col_ref, o_ref):
    v = v_ref[...]
    cols = col_ref[...]
    out = []
    for r in range(kd):
        m = jnp.min(v, axis=0, keepdims=True)
        eq = v == m
        col = jnp.min(jnp.where(eq, cols, NPAD), axis=0, keepdims=True)
        if r % dil == 0:
            out.append(col)
        v = jnp.where(cols == col, INF, v)
    o_ref[...] = jnp.concatenate(out, axis=0)


def _phase_c(cand, cand_cols, kd, dil):
    ncand = cand.shape[0]
    import functools
    body = functools.partial(_phase_c_body, ncand, kd, dil)
    return pl.pallas_call(
        body,
        grid=(NPAD // _RT,),
        in_specs=[
            pl.BlockSpec((ncand, _RT), lambda i: (0, i)),
            pl.BlockSpec((ncand, _RT), lambda i: (0, i)),
        ],
        out_specs=pl.BlockSpec((K, _RT), lambda i: (0, i)),
        out_shape=jax.ShapeDtypeStruct((K, NPAD), jnp.int32),
    )(cand, cand_cols)


def _knn_idx(h, x2, batch_pad, dilation):
    kd = (K - 1) * dilation + 1
    h_pad = jnp.pad(h, ((0, NPAD - N), (0, 0)))
    x2_pad = jnp.pad(x2, (0, NPAD - N))
    dt, cmin = _gram_chunkmin(h_pad, x2_pad, batch_pad)
    ids = _phase_a(cmin, kd)[:kd]
    cand_cols = (ids[:, None, :] * CHUNK
                 + jnp.arange(CHUNK, dtype=jnp.int32)[None, :, None]
                 ).reshape(kd * CHUNK, NPAD)
    cand = jnp.take_along_axis(dt, cand_cols, axis=0)
    idx_t = _phase_c(cand, cand_cols, kd, dilation)
    return idx_t[:, :N].T


def _tail_body(f0_ref, h1_ref, h2_ref, h3_ref,
               wfus_ref, bfus_ref, wp1a_ref, wp1b_ref, bp1_ref,
               wp2_ref, bp2_ref, wp3_ref, bp3_ref, out_ref):
    feats = jnp.concatenate(
        [f0_ref[...], h1_ref[...], h2_ref[...], h3_ref[...]], axis=1)
    z = jax.lax.dot_general(feats, wfus_ref[...], (((1,), (0,)), ((), ())),
                            preferred_element_type=jnp.float32) + bfus_ref[...]
    fus = jnp.max(jax.nn.relu(z), axis=1, keepdims=True)
    hp = jax.lax.dot_general(feats, wp1a_ref[...], (((1,), (0,)), ((), ())),
                             preferred_element_type=jnp.float32)
    hp = jax.nn.relu(hp + fus * wp1b_ref[...] + bp1_ref[...])
    hp = jax.nn.relu(
        jax.lax.dot_general(hp, wp2_ref[...], (((1,), (0,)), ((), ())),
                            preferred_element_type=jnp.float32) + bp2_ref[...])
    out_ref[...] = jax.lax.dot_general(
        hp, wp3_ref[...], (((1,), (0,)), ((), ())),
        preferred_element_type=jnp.float32) + bp3_ref[...]


def _tail(f0, h1, h2, h3, W_fus, b_fus, W_p1, b_p1, W_p2, b_p2, W_p3, b_p3):
    wp1a = W_p1[:4 * C]
    wp1b = W_p1[4 * C:]
    wp3 = jnp.zeros((256, 128), jnp.float32).at[:, :NCLS].set(W_p3)
    bp3 = jnp.zeros((1, 128), jnp.float32).at[:, :NCLS].set(b_p3)
    out = pl.pallas_call(
        _tail_body,
        grid=(N // _TT,),
        in_specs=[
            pl.BlockSpec((_TT, C), lambda i: (i, 0)),
            pl.BlockSpec((_TT, C), lambda i: (i, 0)),
            pl.BlockSpec((_TT, C), lambda i: (i, 0)),
            pl.BlockSpec((_TT, C), lambda i: (i, 0)),
            pl.BlockSpec((4 * C, 1024), lambda i: (0, 0)),
            pl.BlockSpec((1, 1024), lambda i: (0, 0)),
            pl.BlockSpec((4 * C, 4 * C), lambda i: (0, 0)),
            pl.BlockSpec((1, 4 * C), lambda i: (0, 0)),
            pl.BlockSpec((1, 4 * C), lambda i: (0, 0)),
            pl.BlockSpec((4 * C, 256), lambda i: (0, 0)),
            pl.BlockSpec((1, 256), lambda i: (0, 0)),
            pl.BlockSpec((256, 128), lambda i: (0, 0)),
            pl.BlockSpec((1, 128), lambda i: (0, 0)),
        ],
        out_specs=pl.BlockSpec((_TT, 128), lambda i: (i, 0)),
        out_shape=jax.ShapeDtypeStruct((N, 128), jnp.float32),
    )(f0, h1, h2, h3, W_fus, b_fus.reshape(1, -1), wp1a, wp1b,
      b_p1.reshape(1, -1), W_p2, b_p2.reshape(1, -1), wp3, bp3)
    return out[:, :NCLS]


def _pad_mult(v, m):
    r = v.shape[0] % m
    return v if r == 0 else jnp.pad(v, (0, m - r))


def kernel(x, edge_index, batch, W_head, b_head, W1, b1, W2, b2, W3, b3,
           W_fus, b_fus, W_p1, b_p1, W_p2, b_p2, W_p3, b_p3):
    src, dst = edge_index[0], edge_index[1]
    E = src.shape[0]
    both = _pad_mult(jnp.concatenate([dst, src]), _SC_NW * _SC_CHUNK * _SC_NBUF)
    rows = _sc_gather_rows(x, both)
    m = _edge_m(rows[:E], rows[E:2 * E], W_head, b_head)
    agg = jax.ops.segment_max(m, dst, num_segments=N)
    f0 = jnp.where(jnp.isfinite(agg), agg, 0.0)

    batch_pad = jnp.pad(batch, (0, NPAD - N), constant_values=-1)
    h = f0
    feats = [f0]
    for i, (W, b) in enumerate([(W1, b1), (W2, b2), (W3, b3)]):
        dilation = i % 8 + 1
        x2 = jnp.sum(h * h, axis=1)
        idx = _knn_idx(h, x2, batch_pad, dilation)
        idxf = _pad_mult(idx.reshape(-1), _SC_NW * _SC_CHUNK * _SC_NBUF)
        xj = _sc_gather_rows(h, idxf)[:N * K]
        h = _dense_conv(h, xj, W, b)
        feats.append(h)

    return _tail(feats[0], feats[1], feats[2], feats[3],
                 W_fus, b_fus, W_p1, b_p1, W_p2, b_p2, W_p3, b_p3)
